# Initial kernel scaffold; baseline (speedup 1.0000x reference)
import jax
import jax.numpy as jnp
from jax.experimental import pallas as pl


def kernel(user_id, episodes, popularity, year, studio, user_table, episodes_table, popularity_table, year_table, studio_table):
    raise NotImplementedError("write your pallas kernel here")


# speedup vs baseline: 1.6247x; 1.6247x over previous
import functools

import jax
import jax.numpy as jnp
from jax import lax
from jax.experimental import pallas as pl
from jax.experimental.pallas import tpu as pltpu
from jax.experimental.pallas import tpu_sc as plsc

EMBED = 64
NUM_FIELDS = 5
BATCH = 16384
CHUNK = 128


def kernel(user_id, episodes, popularity, year, studio,
           user_table, episodes_table, popularity_table, year_table, studio_table):
    info = plsc.get_sparse_core_info()
    num_workers = info.num_cores * info.num_subcores
    b_per_w = BATCH // num_workers
    n_chunks = b_per_w // CHUNK
    total_chunks = NUM_FIELDS * n_chunks

    mesh = plsc.VectorSubcoreMesh(core_axis_name="c", subcore_axis_name="s")

    @functools.partial(
        pl.kernel,
        mesh=mesh,
        out_type=jax.ShapeDtypeStruct((BATCH, NUM_FIELDS * EMBED), jnp.float32),
        scratch_types=[
            pltpu.VMEM((total_chunks, CHUNK), jnp.int32),
            pltpu.VMEM((2, CHUNK, EMBED), jnp.float32),
            pltpu.SemaphoreType.DMA,
        ],
        compiler_params=pltpu.CompilerParams(use_tc_tiling_on_sc=False),
    )
    def run(uid, ep, pop, yr, st, ut, et, pt, yt, stt, out, idx_v, rows_v, gsem):
        wid = lax.axis_index("s") * info.num_cores + lax.axis_index("c")
        base = wid * b_per_w
        idx_hbm = [uid, ep, pop, yr, st]
        tables = [ut, et, pt, yt, stt]

        for t in range(NUM_FIELDS):
            for c in range(n_chunks):
                pltpu.sync_copy(
                    idx_hbm[t].at[pl.ds(base + c * CHUNK, CHUNK)],
                    idx_v.at[t * n_chunks + c],
                )

        def field_chunk(j):
            return j // n_chunks, j % n_chunks

        def start_gather(j, buf):
            t, _c = field_chunk(j)
            return pltpu.async_copy(
                tables[t].at[idx_v.at[j]], rows_v.at[buf], gsem)

        def write_out(j, buf):
            t, c = field_chunk(j)
            pltpu.sync_copy(
                rows_v.at[buf],
                out.at[pl.ds(base + c * CHUNK, CHUNK),
                       pl.ds(t * EMBED, EMBED)],
            )

        cp = start_gather(0, 0)
        for j in range(total_chunks):
            cp.wait()
            if j + 1 < total_chunks:
                nxt = start_gather(j + 1, (j + 1) % 2)
            write_out(j, j % 2)
            if j + 1 < total_chunks:
                cp = nxt

    return run(user_id, episodes, popularity, year, studio,
               user_table, episodes_table, popularity_table, year_table,
               studio_table)


# speedup vs baseline: 1.7748x; 1.0924x over previous
import functools

import jax
import jax.numpy as jnp
from jax import lax
from jax.experimental import pallas as pl
from jax.experimental.pallas import tpu as pltpu
from jax.experimental.pallas import tpu_sc as plsc

EMBED = 64
NUM_FIELDS = 5
BATCH = 16384
CHUNK = 512


def kernel(user_id, episodes, popularity, year, studio,
           user_table, episodes_table, popularity_table, year_table, studio_table):
    info = plsc.get_sparse_core_info()
    num_workers = info.num_cores * info.num_subcores
    b_per_w = BATCH // num_workers
    n_chunks = b_per_w // CHUNK
    total_chunks = NUM_FIELDS * n_chunks

    mesh = plsc.VectorSubcoreMesh(core_axis_name="c", subcore_axis_name="s")

    @functools.partial(
        pl.kernel,
        mesh=mesh,
        out_type=jax.ShapeDtypeStruct((BATCH, NUM_FIELDS * EMBED), jnp.float32),
        scratch_types=[
            pltpu.VMEM((total_chunks, CHUNK), jnp.int32),
            pltpu.VMEM((2, CHUNK, EMBED), jnp.float32),
            pltpu.SemaphoreType.DMA,
        ],
        compiler_params=pltpu.CompilerParams(use_tc_tiling_on_sc=False),
    )
    def run(uid, ep, pop, yr, st, ut, et, pt, yt, stt, out, idx_v, rows_v, gsem):
        wid = lax.axis_index("s") * info.num_cores + lax.axis_index("c")
        base = wid * b_per_w
        idx_hbm = [uid, ep, pop, yr, st]
        tables = [ut, et, pt, yt, stt]

        for t in range(NUM_FIELDS):
            for c in range(n_chunks):
                pltpu.sync_copy(
                    idx_hbm[t].at[pl.ds(base + c * CHUNK, CHUNK)],
                    idx_v.at[t * n_chunks + c],
                )

        def field_chunk(j):
            return j // n_chunks, j % n_chunks

        def start_gather(j, buf):
            t, _c = field_chunk(j)
            return pltpu.async_copy(
                tables[t].at[idx_v.at[j]], rows_v.at[buf], gsem)

        def write_out(j, buf):
            t, c = field_chunk(j)
            pltpu.sync_copy(
                rows_v.at[buf],
                out.at[pl.ds(base + c * CHUNK, CHUNK),
                       pl.ds(t * EMBED, EMBED)],
            )

        cp = start_gather(0, 0)
        for j in range(total_chunks):
            cp.wait()
            if j + 1 < total_chunks:
                nxt = start_gather(j + 1, (j + 1) % 2)
            write_out(j, j % 2)
            if j + 1 < total_chunks:
                cp = nxt

    return run(user_id, episodes, popularity, year, studio,
               user_table, episodes_table, popularity_table, year_table,
               studio_table)
